# baseline (speedup 1.0000x reference)
import functools

import jax
import jax.numpy as jnp
from jax import lax
from jax.experimental import pallas as pl
from jax.experimental.pallas import tpu as pltpu
from jax.experimental.pallas import tpu_sc as plsc

_SR = 16000.0
_WTLEN = 512
_BLOCK = 160
_NFIXED = 4
_NTILES = 32
_LANES = 16
_RPT = 32
_VPR = _BLOCK // _LANES


def _tc_prep_body(pitch_ref, w_ref, att_ref, m2_ref, *, nblocks, wtlen):
    w = w_ref[...]
    rid = lax.broadcasted_iota(jnp.int32, w.shape, 0)
    proc = jnp.where(rid < _NFIXED, w, jnp.tanh(w))
    a = att_ref[...]
    a = a - jnp.max(a, axis=0, keepdims=True)
    e = jnp.exp(a)
    att = e / jnp.sum(e, axis=0, keepdims=True)
    m2_ref[pl.ds(0, nblocks), :] = lax.dot_general(
        att, proc, (((0,), (0,)), ((), ())),
        preferred_element_type=jnp.float32,
        precision=lax.Precision.HIGHEST,
    )
    inc = pitch_ref[...] / _SR * _WTLEN
    bsum = jnp.sum(inc, axis=1, keepdims=True)
    wi = lax.broadcasted_iota(jnp.int32, (_NTILES, nblocks), 0)
    bi = lax.broadcasted_iota(jnp.int32, (_NTILES, nblocks), 1)
    start = jnp.minimum(wi * _RPT, nblocks - _RPT)
    sel = (bi < start).astype(jnp.float32)
    pref = lax.dot_general(
        sel, bsum, (((1,), (0,)), ((), ())),
        preferred_element_type=jnp.float32,
        precision=lax.Precision.HIGHEST,
    )
    pref_t = jnp.transpose(pref)
    wj = lax.broadcasted_iota(jnp.int32, (_NTILES, wtlen), 0)
    cj = lax.broadcasted_iota(jnp.int32, (_NTILES, wtlen), 1)
    onehot = (cj == wj).astype(jnp.float32)
    car_row = lax.dot_general(
        pref_t, onehot, (((1,), (0,)), ((), ())),
        preferred_element_type=jnp.float32,
        precision=lax.Precision.HIGHEST,
    )
    m2_ref[pl.ds(nblocks, 8), :] = jnp.broadcast_to(car_row, (8, wtlen))


def _sc_synth_body(pitch_hbm, amp_hbm, m2_hbm, out_hbm,
                   pitch_v, amp_v, out_v, m_v, car_v, cs_v, vcar_v,
                   sem_p, sem_a, sem_c, sem_m,
                   *, nblocks, wtlen):
    chunk = _RPT * _BLOCK
    wid = lax.axis_index("s") * 2 + lax.axis_index("c")
    row0 = jnp.minimum(wid * _RPT, nblocks - _RPT)
    t0 = row0 * _BLOCK
    cp_p = pltpu.async_copy(pitch_hbm.at[pl.ds(t0, chunk)], pitch_v, sem_p)
    cp_a = pltpu.async_copy(amp_hbm.at[pl.ds(t0, chunk)], amp_v, sem_a)
    cp_c = pltpu.async_copy(m2_hbm.at[pl.ds(nblocks * wtlen, wtlen)], car_v,
                            sem_c)
    cp_m = pltpu.async_copy(m2_hbm.at[pl.ds(row0 * wtlen, _RPT * wtlen)], m_v,
                            sem_m)

    lane = lax.iota(jnp.int32, _LANES)
    zeros = jnp.zeros((_LANES,), jnp.float32)
    vec_wtlen_f = jnp.full((_LANES,), float(_WTLEN), jnp.float32)
    vec_mask_i = jnp.full((_LANES,), _WTLEN - 1, jnp.int32)
    vec_one_i = jnp.full((_LANES,), 1, jnp.int32)
    sr_v = jnp.full((_LANES,), _SR, jnp.float32)
    wt_f = jnp.full((_LANES,), float(_WTLEN), jnp.float32)
    last_idx = jnp.full((_LANES,), _LANES - 1, jnp.int32)

    def _take(v, idx):
        return lax.gather(
            v, idx[:, None],
            dimension_numbers=lax.GatherDimensionNumbers(
                offset_dims=(), collapsed_slice_dims=(0,),
                start_index_map=(0,)),
            slice_sizes=(1,),
            mode=lax.GatherScatterMode.PROMISE_IN_BOUNDS)

    def _lane_cumsum(x):
        y = x
        for k in (1, 2, 4, 8):
            g = _take(y, jnp.maximum(lane - k, 0))
            y = y + jnp.where(lane >= k, g, zeros)
        return y

    cp_p.wait()

    @plsc.parallel_loop(0, _RPT, unroll=2)
    def _phase_a(r):
        for c in range(_VPR):
            p = pitch_v[pl.ds(r * _BLOCK + c * _LANES, _LANES)]
            inc = p / sr_v * wt_f
            cs_v[pl.ds(r * _BLOCK + c * _LANES, _LANES)] = _lane_cumsum(inc)

    cp_c.wait()
    sub = (wid // _LANES) * _LANES
    carw = _take(car_v[pl.ds(sub, _LANES)],
                 jnp.full((_LANES,), wid - sub, jnp.int32))

    def _phase_b(r, carry):
        for c in range(_VPR):
            vcar_v[pl.ds(r * _BLOCK + c * _LANES, _LANES)] = carry
            cs = cs_v[pl.ds(r * _BLOCK + c * _LANES, _LANES)]
            carry = _take(cs, last_idx) + carry
        return carry

    lax.fori_loop(0, _RPT, _phase_b, carw)

    cp_a.wait()
    cp_m.wait()

    @plsc.parallel_loop(0, _RPT, unroll=2)
    def _phase_c(r):
        base = jnp.full((_LANES,), r * wtlen, jnp.int32)
        for c in range(_VPR):
            o = r * _BLOCK + c * _LANES
            p = pitch_v[pl.ds(o, _LANES)]
            inc = p / sr_v * wt_f
            cs = cs_v[pl.ds(o, _LANES)]
            carry = vcar_v[pl.ds(o, _LANES)]
            idx = cs + carry - inc
            idxm = lax.rem(idx, vec_wtlen_f)
            il = idxm.astype(jnp.int32)
            alpha = idxm - il.astype(jnp.float32)
            ih = (il + vec_one_i) & vec_mask_i
            lo = plsc.load_gather(m_v, [base + il])
            hi = plsc.load_gather(m_v, [base + ih])
            amp = amp_v[pl.ds(o, _LANES)]
            out_v[pl.ds(o, _LANES)] = (lo + alpha * (hi - lo)) * amp

    pltpu.sync_copy(out_v, out_hbm.at[pl.ds(t0, chunk)])


@jax.jit
def kernel(pitch, amplitude, W, attention, sec):
    t = pitch.shape[1]
    nwt, wtlen = W.shape
    nblocks = attention.shape[1]
    chunk = _RPT * _BLOCK

    pitch2 = pitch.reshape(nblocks, _BLOCK)

    m2 = pl.pallas_call(
        functools.partial(_tc_prep_body, nblocks=nblocks, wtlen=wtlen),
        out_shape=jax.ShapeDtypeStruct((nblocks + 8, wtlen), jnp.float32),
    )(pitch2, W, attention)

    mesh = plsc.VectorSubcoreMesh(core_axis_name="c", subcore_axis_name="s")
    sc = functools.partial(
        pl.kernel,
        mesh=mesh,
        compiler_params=pltpu.CompilerParams(needs_layout_passes=False),
        out_type=jax.ShapeDtypeStruct((t,), jnp.float32),
        scratch_types=[
            pltpu.VMEM((chunk,), jnp.float32),
            pltpu.VMEM((chunk,), jnp.float32),
            pltpu.VMEM((chunk,), jnp.float32),
            pltpu.VMEM((_RPT * wtlen,), jnp.float32),
            pltpu.VMEM((wtlen,), jnp.float32),
            pltpu.VMEM((chunk,), jnp.float32),
            pltpu.VMEM((chunk,), jnp.float32),
            pltpu.SemaphoreType.DMA,
            pltpu.SemaphoreType.DMA,
            pltpu.SemaphoreType.DMA,
            pltpu.SemaphoreType.DMA,
        ],
    )(functools.partial(_sc_synth_body, nblocks=nblocks, wtlen=wtlen))
    out = sc(pitch.reshape(t), amplitude.reshape(t), m2.reshape(-1))

    return out.reshape(1, t, 1)


# speedup vs baseline: 1.0715x; 1.0715x over previous
import functools

import jax
import jax.numpy as jnp
from jax import lax
from jax.experimental import pallas as pl
from jax.experimental.pallas import tpu as pltpu
from jax.experimental.pallas import tpu_sc as plsc

_SR = 16000.0
_WTLEN = 512
_BLOCK = 160
_NFIXED = 4
_NTILES = 32
_LANES = 16
_RPT = 32
_VPR = _BLOCK // _LANES


def _tc_prep_body(pitch_ref, w_ref, att_ref, m2_ref, *, nblocks, wtlen):
    w = w_ref[...]
    rid = lax.broadcasted_iota(jnp.int32, w.shape, 0)
    proc = jnp.where(rid < _NFIXED, w, jnp.tanh(w))
    a = att_ref[...]
    a = a - jnp.max(a, axis=0, keepdims=True)
    e = jnp.exp(a)
    att = e / jnp.sum(e, axis=0, keepdims=True)
    m2_ref[pl.ds(0, nblocks), :] = lax.dot_general(
        att, proc, (((0,), (0,)), ((), ())),
        preferred_element_type=jnp.float32,
        precision=lax.Precision.HIGHEST,
    )
    inc = pitch_ref[...] / _SR * _WTLEN
    bsum = jnp.sum(inc, axis=1, keepdims=True)
    wi = lax.broadcasted_iota(jnp.int32, (_NTILES, nblocks), 0)
    bi = lax.broadcasted_iota(jnp.int32, (_NTILES, nblocks), 1)
    start = jnp.minimum(wi * _RPT, nblocks - _RPT)
    sel = (bi < start).astype(jnp.float32)
    pref = lax.dot_general(
        sel, bsum, (((1,), (0,)), ((), ())),
        preferred_element_type=jnp.float32,
        precision=lax.Precision.HIGHEST,
    )
    pref_t = jnp.transpose(pref)
    wj = lax.broadcasted_iota(jnp.int32, (_NTILES, wtlen), 0)
    cj = lax.broadcasted_iota(jnp.int32, (_NTILES, wtlen), 1)
    onehot = (cj == wj).astype(jnp.float32)
    car_row = lax.dot_general(
        pref_t, onehot, (((1,), (0,)), ((), ())),
        preferred_element_type=jnp.float32,
        precision=lax.Precision.HIGHEST,
    )
    m2_ref[pl.ds(nblocks, 8), :] = jnp.broadcast_to(car_row, (8, wtlen))


def _sc_synth_body(pitch_hbm, amp_hbm, m2_hbm, out_hbm,
                   pitch_v, amp_v, out_v, m_v, car_v, vcar_v,
                   sem_p, sem_a, sem_c, sem_m,
                   *, nblocks, wtlen):
    chunk = _RPT * _BLOCK
    wid = lax.axis_index("s") * 2 + lax.axis_index("c")
    row0 = jnp.minimum(wid * _RPT, nblocks - _RPT)
    t0 = row0 * _BLOCK
    cp_p = pltpu.async_copy(pitch_hbm.at[pl.ds(t0, chunk)], pitch_v, sem_p)
    cp_a = pltpu.async_copy(amp_hbm.at[pl.ds(t0, chunk)], amp_v, sem_a)
    cp_c = pltpu.async_copy(m2_hbm.at[pl.ds(nblocks * wtlen, wtlen)], car_v,
                            sem_c)
    cp_m = pltpu.async_copy(m2_hbm.at[pl.ds(row0 * wtlen, _RPT * wtlen)], m_v,
                            sem_m)

    lane = lax.iota(jnp.int32, _LANES)
    zeros = jnp.zeros((_LANES,), jnp.float32)
    vec_wtlen_f = jnp.full((_LANES,), float(_WTLEN), jnp.float32)
    vec_mask_i = jnp.full((_LANES,), _WTLEN - 1, jnp.int32)
    vec_one_i = jnp.full((_LANES,), 1, jnp.int32)
    sr_v = jnp.full((_LANES,), _SR, jnp.float32)
    wt_f = jnp.full((_LANES,), float(_WTLEN), jnp.float32)
    last_idx = jnp.full((_LANES,), _LANES - 1, jnp.int32)

    def _take(v, idx):
        return lax.gather(
            v, idx[:, None],
            dimension_numbers=lax.GatherDimensionNumbers(
                offset_dims=(), collapsed_slice_dims=(0,),
                start_index_map=(0,)),
            slice_sizes=(1,),
            mode=lax.GatherScatterMode.PROMISE_IN_BOUNDS)

    def _lane_cumsum(x):
        y = x
        for k in (1, 2, 4, 8):
            g = _take(y, jnp.maximum(lane - k, 0))
            y = y + jnp.where(lane >= k, g, zeros)
        return y

    nvec = chunk // _LANES
    cp_p.wait()

    @plsc.parallel_loop(0, nvec, unroll=8)
    def _phase_a(i):
        p = pitch_v[pl.ds(i * _LANES, _LANES)]
        inc = p / sr_v * wt_f
        out_v[pl.ds(i * _LANES, _LANES)] = _lane_cumsum(inc)

    cp_c.wait()
    sub = (wid // _LANES) * _LANES
    carw = _take(car_v[pl.ds(sub, _LANES)],
                 jnp.full((_LANES,), wid - sub, jnp.int32))

    def _phase_b(i, carry):
        vcar_v[pl.ds(i * _LANES, _LANES)] = carry
        cs = out_v[pl.ds(i * _LANES, _LANES)]
        return _take(cs, last_idx) + carry

    lax.fori_loop(0, nvec, _phase_b, carw)

    cp_a.wait()
    cp_m.wait()

    @plsc.parallel_loop(0, nvec, unroll=4)
    def _phase_c(i):
        o = i * _LANES
        p = pitch_v[pl.ds(o, _LANES)]
        inc = p / sr_v * wt_f
        cs = out_v[pl.ds(o, _LANES)]
        carry = vcar_v[pl.ds(o, _LANES)]
        idx = cs + carry - inc
        idxm = lax.rem(idx, vec_wtlen_f)
        il = idxm.astype(jnp.int32)
        alpha = idxm - il.astype(jnp.float32)
        ih = (il + vec_one_i) & vec_mask_i
        base = jnp.full((_LANES,), (i // _VPR) * wtlen, jnp.int32)
        lo = plsc.load_gather(m_v, [base + il])
        hi = plsc.load_gather(m_v, [base + ih])
        amp = amp_v[pl.ds(o, _LANES)]
        out_v[pl.ds(o, _LANES)] = (lo + alpha * (hi - lo)) * amp

    pltpu.sync_copy(out_v, out_hbm.at[pl.ds(t0, chunk)])


@jax.jit
def kernel(pitch, amplitude, W, attention, sec):
    t = pitch.shape[1]
    nwt, wtlen = W.shape
    nblocks = attention.shape[1]
    chunk = _RPT * _BLOCK

    pitch2 = pitch.reshape(nblocks, _BLOCK)

    m2 = pl.pallas_call(
        functools.partial(_tc_prep_body, nblocks=nblocks, wtlen=wtlen),
        out_shape=jax.ShapeDtypeStruct((nblocks + 8, wtlen), jnp.float32),
    )(pitch2, W, attention)

    mesh = plsc.VectorSubcoreMesh(core_axis_name="c", subcore_axis_name="s")
    sc = functools.partial(
        pl.kernel,
        mesh=mesh,
        compiler_params=pltpu.CompilerParams(needs_layout_passes=False),
        out_type=jax.ShapeDtypeStruct((t,), jnp.float32),
        scratch_types=[
            pltpu.VMEM((chunk,), jnp.float32),
            pltpu.VMEM((chunk,), jnp.float32),
            pltpu.VMEM((chunk,), jnp.float32),
            pltpu.VMEM((_RPT * wtlen,), jnp.float32),
            pltpu.VMEM((wtlen,), jnp.float32),
            pltpu.VMEM((chunk,), jnp.float32),
            pltpu.SemaphoreType.DMA,
            pltpu.SemaphoreType.DMA,
            pltpu.SemaphoreType.DMA,
            pltpu.SemaphoreType.DMA,
        ],
    )(functools.partial(_sc_synth_body, nblocks=nblocks, wtlen=wtlen))
    out = sc(pitch.reshape(t), amplitude.reshape(t), m2.reshape(-1))

    return out.reshape(1, t, 1)


# speedup vs baseline: 1.0833x; 1.0110x over previous
import functools

import jax
import jax.numpy as jnp
from jax import lax
from jax.experimental import pallas as pl
from jax.experimental.pallas import tpu as pltpu
from jax.experimental.pallas import tpu_sc as plsc

_SR = 16000.0
_WTLEN = 512
_BLOCK = 160
_NFIXED = 4
_NTILES = 32
_LANES = 16
_RPT = 32
_VPR = _BLOCK // _LANES


def _tc_prep_body(pitch_ref, w_ref, att_ref, m2_ref, *, nblocks, wtlen):
    w = w_ref[...]
    rid = lax.broadcasted_iota(jnp.int32, w.shape, 0)
    proc = jnp.where(rid < _NFIXED, w, jnp.tanh(w))
    a = att_ref[...]
    a = a - jnp.max(a, axis=0, keepdims=True)
    e = jnp.exp(a)
    att = e / jnp.sum(e, axis=0, keepdims=True)
    m2_ref[pl.ds(0, nblocks), :] = lax.dot_general(
        att, proc, (((0,), (0,)), ((), ())),
        preferred_element_type=jnp.float32,
        precision=lax.Precision.HIGHEST,
    )
    inc = pitch_ref[...] / _SR * _WTLEN
    bsum = jnp.sum(inc, axis=1, keepdims=True)
    wi = lax.broadcasted_iota(jnp.int32, (_NTILES, nblocks), 0)
    bi = lax.broadcasted_iota(jnp.int32, (_NTILES, nblocks), 1)
    start = jnp.minimum(wi * _RPT, nblocks - _RPT)
    sel = (bi < start).astype(jnp.float32)
    pref = lax.dot_general(
        sel, bsum, (((1,), (0,)), ((), ())),
        preferred_element_type=jnp.float32,
        precision=lax.Precision.HIGHEST,
    )
    pref_t = jnp.transpose(pref)
    wj = lax.broadcasted_iota(jnp.int32, (_NTILES, wtlen), 0)
    cj = lax.broadcasted_iota(jnp.int32, (_NTILES, wtlen), 1)
    onehot = (cj == wj).astype(jnp.float32)
    car_row = lax.dot_general(
        pref_t, onehot, (((1,), (0,)), ((), ())),
        preferred_element_type=jnp.float32,
        precision=lax.Precision.HIGHEST,
    )
    m2_ref[pl.ds(nblocks, 8), :] = jnp.broadcast_to(car_row, (8, wtlen))


def _sc_synth_body(pitch_hbm, amp_hbm, m2_hbm, out_hbm,
                   pitch_v, amp_v, out_v, m_v, car_v, vcar_v,
                   rowtot_v, rowcar_v,
                   sem_p, sem_a, sem_c, sem_m,
                   *, nblocks, wtlen):
    chunk = _RPT * _BLOCK
    wid = lax.axis_index("s") * 2 + lax.axis_index("c")
    row0 = jnp.minimum(wid * _RPT, nblocks - _RPT)
    t0 = row0 * _BLOCK
    cp_p = pltpu.async_copy(pitch_hbm.at[pl.ds(t0, chunk)], pitch_v, sem_p)
    cp_a = pltpu.async_copy(amp_hbm.at[pl.ds(t0, chunk)], amp_v, sem_a)
    cp_c = pltpu.async_copy(m2_hbm.at[pl.ds(nblocks * wtlen, wtlen)], car_v,
                            sem_c)
    cp_m = pltpu.async_copy(m2_hbm.at[pl.ds(row0 * wtlen, _RPT * wtlen)], m_v,
                            sem_m)

    lane = lax.iota(jnp.int32, _LANES)
    zeros = jnp.zeros((_LANES,), jnp.float32)
    vec_wtlen_f = jnp.full((_LANES,), float(_WTLEN), jnp.float32)
    vec_mask_i = jnp.full((_LANES,), _WTLEN - 1, jnp.int32)
    vec_one_i = jnp.full((_LANES,), 1, jnp.int32)
    sr_v = jnp.full((_LANES,), _SR, jnp.float32)
    wt_f = jnp.full((_LANES,), float(_WTLEN), jnp.float32)
    last_idx = jnp.full((_LANES,), _LANES - 1, jnp.int32)

    def _take(v, idx):
        return lax.gather(
            v, idx[:, None],
            dimension_numbers=lax.GatherDimensionNumbers(
                offset_dims=(), collapsed_slice_dims=(0,),
                start_index_map=(0,)),
            slice_sizes=(1,),
            mode=lax.GatherScatterMode.PROMISE_IN_BOUNDS)

    def _lane_cumsum(x):
        y = x
        for k in (1, 2, 4, 8):
            g = _take(y, jnp.maximum(lane - k, 0))
            y = y + jnp.where(lane >= k, g, zeros)
        return y

    nvec = chunk // _LANES
    cp_p.wait()

    @plsc.parallel_loop(0, nvec, unroll=8)
    def _phase_a(i):
        p = pitch_v[pl.ds(i * _LANES, _LANES)]
        inc = p / sr_v * wt_f
        out_v[pl.ds(i * _LANES, _LANES)] = _lane_cumsum(inc)

    @plsc.parallel_loop(0, _RPT, unroll=2)
    def _phase_b1(r):
        tot = zeros
        for c in range(_VPR):
            cs = out_v[pl.ds(r * _BLOCK + c * _LANES, _LANES)]
            tot = tot + _take(cs, last_idx)
        rowtot_v[pl.ds(r * _LANES, _LANES)] = tot

    cp_c.wait()
    sub = (wid // _LANES) * _LANES
    carw = _take(car_v[pl.ds(sub, _LANES)],
                 jnp.full((_LANES,), wid - sub, jnp.int32))

    def _phase_b2(r, carry):
        rowcar_v[pl.ds(r * _LANES, _LANES)] = carry
        return carry + rowtot_v[pl.ds(r * _LANES, _LANES)]

    lax.fori_loop(0, _RPT, _phase_b2, carw)

    @plsc.parallel_loop(0, _RPT, unroll=2)
    def _phase_b3(r):
        carry = rowcar_v[pl.ds(r * _LANES, _LANES)]
        for c in range(_VPR):
            vcar_v[pl.ds(r * _BLOCK + c * _LANES, _LANES)] = carry
            cs = out_v[pl.ds(r * _BLOCK + c * _LANES, _LANES)]
            carry = carry + _take(cs, last_idx)

    cp_a.wait()
    cp_m.wait()

    @plsc.parallel_loop(0, nvec, unroll=4)
    def _phase_c(i):
        o = i * _LANES
        p = pitch_v[pl.ds(o, _LANES)]
        inc = p / sr_v * wt_f
        cs = out_v[pl.ds(o, _LANES)]
        carry = vcar_v[pl.ds(o, _LANES)]
        idx = cs + carry - inc
        idxm = lax.rem(idx, vec_wtlen_f)
        il = idxm.astype(jnp.int32)
        alpha = idxm - il.astype(jnp.float32)
        ih = (il + vec_one_i) & vec_mask_i
        base = jnp.full((_LANES,), (i // _VPR) * wtlen, jnp.int32)
        lo = plsc.load_gather(m_v, [base + il])
        hi = plsc.load_gather(m_v, [base + ih])
        amp = amp_v[pl.ds(o, _LANES)]
        out_v[pl.ds(o, _LANES)] = (lo + alpha * (hi - lo)) * amp

    pltpu.sync_copy(out_v, out_hbm.at[pl.ds(t0, chunk)])


@jax.jit
def kernel(pitch, amplitude, W, attention, sec):
    t = pitch.shape[1]
    nwt, wtlen = W.shape
    nblocks = attention.shape[1]
    chunk = _RPT * _BLOCK

    pitch2 = pitch.reshape(nblocks, _BLOCK)

    m2 = pl.pallas_call(
        functools.partial(_tc_prep_body, nblocks=nblocks, wtlen=wtlen),
        out_shape=jax.ShapeDtypeStruct((nblocks + 8, wtlen), jnp.float32),
    )(pitch2, W, attention)

    mesh = plsc.VectorSubcoreMesh(core_axis_name="c", subcore_axis_name="s")
    sc = functools.partial(
        pl.kernel,
        mesh=mesh,
        compiler_params=pltpu.CompilerParams(needs_layout_passes=False),
        out_type=jax.ShapeDtypeStruct((t,), jnp.float32),
        scratch_types=[
            pltpu.VMEM((chunk,), jnp.float32),
            pltpu.VMEM((chunk,), jnp.float32),
            pltpu.VMEM((chunk,), jnp.float32),
            pltpu.VMEM((_RPT * wtlen,), jnp.float32),
            pltpu.VMEM((wtlen,), jnp.float32),
            pltpu.VMEM((chunk,), jnp.float32),
            pltpu.VMEM((_RPT * _LANES,), jnp.float32),
            pltpu.VMEM((_RPT * _LANES,), jnp.float32),
            pltpu.SemaphoreType.DMA,
            pltpu.SemaphoreType.DMA,
            pltpu.SemaphoreType.DMA,
            pltpu.SemaphoreType.DMA,
        ],
    )(functools.partial(_sc_synth_body, nblocks=nblocks, wtlen=wtlen))
    out = sc(pitch.reshape(t), amplitude.reshape(t), m2.reshape(-1))

    return out.reshape(1, t, 1)
